# baseline (speedup 1.0000x reference)
import jax
import jax.numpy as jnp
from jax.experimental import pallas as pl


def _copy_body(x_ref, p_ref, x_out, p_out):
    x_out[...] = x_ref[...]
    p_out[...] = p_ref[...]


def kernel(x_unpack, pos_unpack, edge_index):
    x_out, p_out = pl.pallas_call(
        _copy_body,
        out_shape=(
            jax.ShapeDtypeStruct(x_unpack.shape, x_unpack.dtype),
            jax.ShapeDtypeStruct(pos_unpack.shape, pos_unpack.dtype),
        ),
    )(x_unpack, pos_unpack)
    return (x_out, p_out)
